# baseline (speedup 1.0000x reference)
import functools

import jax
import jax.numpy as jnp
from jax import lax
from jax.experimental import pallas as pl
from jax.experimental.pallas import tpu as pltpu
from jax.experimental.pallas import tpu_sc as plsc

NC, NS, L = 2, 16, 16
NW = NC * NS

B = 16384 * 26
D = 128
CHUNK = 128
GPB = 2
SUPER = GPB * CHUNK
B_PER_W = B // NW
N_CHUNKS = B_PER_W // CHUNK
N_SUPER = B_PER_W // SUPER
NBUF = 2


def _emb_body(table_hbm, idx_hbm, out_hbm, idx_v, bufs, gsems, osems):
    wid = lax.axis_index("s") * NC + lax.axis_index("c")
    row_base = wid * B_PER_W

    pltpu.sync_copy(idx_hbm.at[wid], idx_v)

    def gathers(s, b):
        return [
            pltpu.make_async_copy(
                table_hbm.at[idx_v.at[s * GPB + g]],
                bufs[b].at[pl.ds(g * CHUNK, CHUNK)],
                gsems[b])
            for g in range(GPB)
        ]

    def writeback(s, b):
        return pltpu.make_async_copy(
            bufs[b], out_hbm.at[pl.ds(row_base + s * SUPER, SUPER)], osems[b])

    for b in range(NBUF):
        for c in gathers(b, b):
            c.start()

    def step(ss, _):
        for b in range(NBUF):
            s = ss * NBUF + b
            for c in gathers(s, b):
                c.wait()
            writeback(s, b).start()
            nxt = s + NBUF

            @pl.when(nxt < N_SUPER)
            def _():
                writeback(s, b).wait()
                for c in gathers(nxt, b):
                    c.start()
        return 0

    lax.fori_loop(0, N_SUPER // NBUF, step, 0)

    for b in range(NBUF):
        s = N_SUPER - NBUF + b
        writeback(s, b).wait()


@jax.jit
def _emb_lookup(idx2d, table):
    mesh = plsc.VectorSubcoreMesh(core_axis_name="c", subcore_axis_name="s")
    f = pl.kernel(
        _emb_body,
        out_type=jax.ShapeDtypeStruct((B, D), jnp.float32),
        mesh=mesh,
        scratch_types=[
            pltpu.VMEM((N_CHUNKS, CHUNK), jnp.int32),
            [pltpu.VMEM((SUPER, D), jnp.float32) for _ in range(NBUF)],
            [pltpu.SemaphoreType.DMA for _ in range(NBUF)],
            [pltpu.SemaphoreType.DMA for _ in range(NBUF)],
        ],
    )
    return f(table, idx2d)


def kernel(node_fea, table):
    idx2d = node_fea.astype(jnp.int32).reshape(NW, N_CHUNKS, CHUNK)
    return _emb_lookup(idx2d, table)


# speedup vs baseline: 1.0259x; 1.0259x over previous
import functools

import jax
import jax.numpy as jnp
from jax import lax
from jax.experimental import pallas as pl
from jax.experimental.pallas import tpu as pltpu
from jax.experimental.pallas import tpu_sc as plsc

NC, NS, L = 2, 16, 16
NW = NC * NS

B = 16384 * 26
D = 128
CHUNK = 104
B_PER_W = B // NW
N_CHUNKS = B_PER_W // CHUNK
NBUF = 8


def _emb_body(table_hbm, idx_hbm, out_hbm, idx_v, bufs, gsems, osems):
    wid = lax.axis_index("s") * NC + lax.axis_index("c")
    row_base = wid * B_PER_W

    pltpu.sync_copy(idx_hbm.at[wid], idx_v)

    def gather(j, b):
        return pltpu.make_async_copy(
            table_hbm.at[idx_v.at[j]], bufs[b], gsems[b])

    def writeback(j, b):
        return pltpu.make_async_copy(
            bufs[b], out_hbm.at[pl.ds(row_base + j * CHUNK, CHUNK)], osems[b])

    for b in range(NBUF):
        gather(b, b).start()

    def step(jj, _):
        for b in range(NBUF):
            j = jj * NBUF + b
            gather(j, b).wait()
            writeback(j, b).start()
            nxt = j + NBUF

            @pl.when(nxt < N_CHUNKS)
            def _():
                writeback(j, b).wait()
                gather(nxt, b).start()
        return 0

    lax.fori_loop(0, N_CHUNKS // NBUF, step, 0)

    for b in range(NBUF):
        j = N_CHUNKS - NBUF + b
        writeback(j, b).wait()


@jax.jit
def _emb_lookup(idx2d, table):
    mesh = plsc.VectorSubcoreMesh(core_axis_name="c", subcore_axis_name="s")
    f = pl.kernel(
        _emb_body,
        out_type=jax.ShapeDtypeStruct((B, D), jnp.float32),
        mesh=mesh,
        scratch_types=[
            pltpu.VMEM((N_CHUNKS, CHUNK), jnp.int32),
            [pltpu.VMEM((CHUNK, D), jnp.float32) for _ in range(NBUF)],
            [pltpu.SemaphoreType.DMA for _ in range(NBUF)],
            [pltpu.SemaphoreType.DMA for _ in range(NBUF)],
        ],
    )
    return f(table, idx2d)


def kernel(node_fea, table):
    idx2d = node_fea.astype(jnp.int32).reshape(NW, N_CHUNKS, CHUNK)
    return _emb_lookup(idx2d, table)
